# Initial kernel scaffold; baseline (speedup 1.0000x reference)
import jax
import jax.numpy as jnp
from jax.experimental import pallas as pl


def kernel(x, edge_index, weight):
    raise NotImplementedError("write your pallas kernel here")


# speedup vs baseline: 1.0649x; 1.0649x over previous
import functools

import jax
import jax.numpy as jnp
from jax import lax
from jax.experimental import pallas as pl
from jax.experimental.pallas import tpu as pltpu

_COMMIT = 0.25


_R = 400
_C = 2048


def _dist_argmin_body(xsq_ref, wsq_ref, x_ref, w_ref, out_ref, mv_ref, mi_ref):
    j = pl.program_id(0)
    i = pl.program_id(1)
    nk = pl.num_programs(0)
    r = x_ref.shape[0]
    c = w_ref.shape[0]

    s = lax.dot_general(x_ref[...], w_ref[...], (((1,), (1,)), ((), ())),
                        preferred_element_type=jnp.float32)
    d = (xsq_ref[...] + wsq_ref[0]) - 2.0 * s
    bmin = jnp.min(d, axis=1, keepdims=True)
    lane = lax.broadcasted_iota(jnp.int32, d.shape, 1)
    cand = jnp.where(d == bmin, lane, c)
    barg = jnp.min(cand, axis=1, keepdims=True) + j * c

    rows = pl.ds(i * r, r)

    @pl.when(j == 0)
    def _():
        mv_ref[rows, :] = jnp.full((r, 1), jnp.inf, jnp.float32)
        mi_ref[rows, :] = jnp.zeros((r, 1), jnp.int32)

    prev_v = mv_ref[rows, :]
    prev_i = mi_ref[rows, :]
    upd = bmin < prev_v
    mv_ref[rows, :] = jnp.where(upd, bmin, prev_v)
    mi_ref[rows, :] = jnp.where(upd, barg, prev_i)

    @pl.when(j == nk - 1)
    def _():
        out_ref[...] = mi_ref[rows, :]


def _dist_argmin(x, weight, xsq, wsq):
    n, d_model = x.shape
    k = weight.shape[0]
    nblk = n // _R
    kblk = k // _C
    wsq3 = wsq.reshape(kblk, 1, _C)
    out = pl.pallas_call(
        _dist_argmin_body,
        grid=(kblk, nblk),
        in_specs=[
            pl.BlockSpec((_R, 1), lambda j, i: (i, 0)),
            pl.BlockSpec((1, 1, _C), lambda j, i: (j, 0, 0)),
            pl.BlockSpec((_R, d_model), lambda j, i: (i, 0)),
            pl.BlockSpec((_C, d_model), lambda j, i: (j, 0)),
        ],
        out_specs=pl.BlockSpec((_R, 1), lambda j, i: (i, 0)),
        out_shape=jax.ShapeDtypeStruct((n, 1), jnp.int32),
        scratch_shapes=[
            pltpu.VMEM((n, 1), jnp.float32),
            pltpu.VMEM((n, 1), jnp.int32),
        ],
    )(xsq, wsq3, x, weight)
    return out.reshape(n)


def kernel(x, edge_index, weight):
    n, d_model = x.shape
    xsq = jnp.sum(x ** 2, axis=1, keepdims=True)
    wsq = jnp.sum(weight ** 2, axis=1)
    idx = _dist_argmin(x, weight, xsq, wsq)

    z_q = jnp.take(weight, idx, axis=0)
    t = z_q - x
    z_q_st = x + t
    m = jnp.mean(t * t)
    loss = m + _COMMIT * m
    return (z_q_st, edge_index, loss, idx)


# speedup vs baseline: 1.2480x; 1.1720x over previous
import functools

import jax
import jax.numpy as jnp
from jax import lax
from jax.experimental import pallas as pl
from jax.experimental.pallas import tpu as pltpu

_COMMIT = 0.25


_R = 400
_C = 2048


def _dist_argmin_body(xsq_ref, wsq_ref, x_ref, w_ref, out_ref):
    s2 = lax.dot_general(x_ref[...], w_ref[...], (((1,), (1,)), ((), ())),
                         preferred_element_type=jnp.float32)
    d = (xsq_ref[...] + wsq_ref[0]) - s2
    di = lax.bitcast_convert_type(d, jnp.int32)
    rel = di - di[:, 0:1]
    lane = lax.broadcasted_iota(jnp.int32, d.shape, 1)
    key = jnp.left_shift(rel, 13) + lane
    kmin = jnp.min(key, axis=1, keepdims=True)
    out_ref[...] = jnp.bitwise_and(kmin, d.shape[1] - 1)


def _dist_argmin(x, weight, xsq, wsq):
    n, d_model = x.shape
    k = weight.shape[0]
    nblk = n // _R
    wsq3 = wsq.reshape(1, 1, k)
    out = pl.pallas_call(
        _dist_argmin_body,
        grid=(nblk,),
        in_specs=[
            pl.BlockSpec((_R, 1), lambda i: (i, 0)),
            pl.BlockSpec((1, 1, k), lambda i: (0, 0, 0)),
            pl.BlockSpec((_R, d_model), lambda i: (i, 0)),
            pl.BlockSpec((k, d_model), lambda i: (0, 0)),
        ],
        out_specs=pl.BlockSpec((_R, 1), lambda i: (i, 0)),
        out_shape=jax.ShapeDtypeStruct((n, 1), jnp.int32),
    )(xsq, wsq3, 2.0 * x, weight)
    return out.reshape(n)


def kernel(x, edge_index, weight):
    n, d_model = x.shape
    xsq = jnp.sum(x ** 2, axis=1, keepdims=True)
    wsq = jnp.sum(weight ** 2, axis=1)
    idx = _dist_argmin(x, weight, xsq, wsq)

    z_q = jnp.take(weight, idx, axis=0)
    t = z_q - x
    z_q_st = x + t
    m = jnp.mean(t * t)
    loss = m + _COMMIT * m
    return (z_q_st, edge_index, loss, idx)
